# baseline (speedup 1.0000x reference)
import functools

import jax
import jax.numpy as jnp
from jax import lax
from jax.experimental import pallas as pl
from jax.experimental.pallas import tpu as pltpu
from jax.experimental.pallas import tpu_sc as plsc


def _transform_pairs(pairs, wd, bd, blk):
    R2, E2 = pairs.shape
    H2 = wd.shape[1]

    def body(x_ref, w_ref, b_ref, out_ref):
        out_ref[...] = (
            jnp.dot(x_ref[...], w_ref[...], preferred_element_type=jnp.float32)
            + b_ref[...]
        )

    return pl.pallas_call(
        body,
        grid=(R2 // blk,),
        in_specs=[
            pl.BlockSpec((blk, E2), lambda i: (i, 0)),
            pl.BlockSpec((E2, H2), lambda i: (0, 0)),
            pl.BlockSpec((1, H2), lambda i: (0, 0)),
        ],
        out_specs=pl.BlockSpec((blk, H2), lambda i: (i, 0)),
        out_shape=jax.ShapeDtypeStruct((R2, H2), jnp.float32),
    )(pairs, wd, bd)


def _sc_lookup_sum(t1, t2, uid, sid, tok, hdim, num_workers, chunk):
    per_w = tok // num_workers
    n_chunks = per_w // chunk
    mesh = plsc.VectorSubcoreMesh(core_axis_name="c", subcore_axis_name="s")
    nc = mesh.num_cores

    @functools.partial(
        pl.kernel,
        out_type=jax.ShapeDtypeStruct((tok, hdim), jnp.float32),
        mesh=mesh,
        scratch_types=[
            pltpu.VMEM((chunk,), jnp.int32),
            pltpu.VMEM((chunk,), jnp.int32),
            pltpu.VMEM((chunk, hdim), jnp.float32),
            pltpu.SemaphoreType.DMA,
            pltpu.SemaphoreType.DMA,
        ],
        compiler_params=pltpu.CompilerParams(use_tc_tiling_on_sc=False),
    )
    def k(t1_hbm, t2_hbm, uid_hbm, sid_hbm, out_hbm, idx1_v, idx2_v, buf, sem1, sem2):
        wid = lax.axis_index("s") * nc + lax.axis_index("c")
        base = wid * per_w

        def chunk_body(i, carry):
            off = base + i * chunk
            pltpu.sync_copy(uid_hbm.at[pl.ds(off, chunk)], idx1_v)
            pltpu.sync_copy(sid_hbm.at[pl.ds(off, chunk)], idx2_v)
            pltpu.async_copy(t1_hbm.at[idx1_v], buf, sem1).wait()
            pltpu.async_copy(t2_hbm.at[idx2_v], buf, sem2, add=True).wait()
            pltpu.sync_copy(buf, out_hbm.at[pl.ds(off, chunk)])
            return carry

        lax.fori_loop(0, n_chunks, chunk_body, 0)

    return k(t1, t2, uid, sid)


def _blockdiag2(w):
    E, H = w.shape
    z = jnp.zeros((E, H), dtype=w.dtype)
    return jnp.concatenate(
        [jnp.concatenate([w, z], axis=1), jnp.concatenate([z, w], axis=1)], axis=0
    )


def kernel(user_id_sequence, skill_sequence, emb1, emb2, W1, b1):
    B, L = user_id_sequence.shape
    E = emb1.shape[1]
    H = W1.shape[1]
    tok = B * L
    r1 = emb1.shape[0]
    r2 = emb2.shape[0]

    e1p = emb1.reshape(r1 // 2, 2 * E)
    e2p = emb2.reshape(r2 // 2, 2 * E)
    wd1 = _blockdiag2(W1[:E])
    wd2 = _blockdiag2(W1[E:])
    bd1 = jnp.concatenate([b1, b1]).reshape(1, 2 * H).astype(jnp.float32)
    bd2 = jnp.zeros((1, 2 * H), dtype=jnp.float32)

    t1 = _transform_pairs(e1p, wd1, bd1, blk=10000).reshape(r1, H)
    t2 = _transform_pairs(e2p, wd2, bd2, blk=10000).reshape(r2, H)

    uid = user_id_sequence.reshape(tok).astype(jnp.int32)
    sid = skill_sequence.reshape(tok).astype(jnp.int32)

    out = _sc_lookup_sum(t1, t2, uid, sid, tok, H, num_workers=32, chunk=512)
    return out.reshape(B, L, H)


# speedup vs baseline: 1.1314x; 1.1314x over previous
import functools

import jax
import jax.numpy as jnp
from jax import lax
from jax.experimental import pallas as pl
from jax.experimental.pallas import tpu as pltpu
from jax.experimental.pallas import tpu_sc as plsc


def _transform_table(emb, w, b, blk2):
    R, E = emb.shape
    H = w.shape[1]
    R2 = R // 2
    n = R2 // blk2

    def body(lo_ref, hi_ref, w_ref, b_ref, out_ref):
        wv = w_ref[...]
        bv = b_ref[...]
        out_ref[:, 0:H] = (
            jnp.dot(lo_ref[...], wv, preferred_element_type=jnp.float32) + bv
        )
        out_ref[:, H : 2 * H] = (
            jnp.dot(hi_ref[...], wv, preferred_element_type=jnp.float32) + bv
        )

    return pl.pallas_call(
        body,
        grid=(n,),
        in_specs=[
            pl.BlockSpec((blk2, E), lambda i: (i, 0)),
            pl.BlockSpec((blk2, E), lambda i: (i + n, 0)),
            pl.BlockSpec((E, H), lambda i: (0, 0)),
            pl.BlockSpec((1, H), lambda i: (0, 0)),
        ],
        out_specs=pl.BlockSpec((blk2, 2 * H), lambda i: (i, 0)),
        out_shape=jax.ShapeDtypeStruct((R2, 2 * H), jnp.float32),
    )(emb, emb, w, b)


def _sc_lookup_sum(t1, t2, uid, sid, r1, r2, B, L, hdim, num_workers, rows_per_chunk):
    tok = B * L
    per_w = tok // num_workers
    rows_w = B // num_workers
    chunk = rows_per_chunk * L
    n_chunks = rows_w // rows_per_chunk
    mesh = plsc.VectorSubcoreMesh(core_axis_name="c", subcore_axis_name="s")
    nc = mesh.num_cores
    r1_half = r1 // 2
    r2_half = r2 // 2

    @functools.partial(
        pl.kernel,
        out_type=jax.ShapeDtypeStruct((B, L, hdim), jnp.float32),
        mesh=mesh,
        scratch_types=[
            pltpu.VMEM((chunk,), jnp.int32),
            pltpu.VMEM((chunk,), jnp.int32),
            pltpu.VMEM((chunk, hdim), jnp.float32),
            pltpu.SemaphoreType.DMA,
            pltpu.SemaphoreType.DMA,
        ],
        compiler_params=pltpu.CompilerParams(use_tc_tiling_on_sc=False),
    )
    def k(t1_hbm, t2_hbm, uid_hbm, sid_hbm, out_hbm, idx1_v, idx2_v, buf, sem1, sem2):
        wid = lax.axis_index("s") * nc + lax.axis_index("c")
        base = wid * per_w
        base_b = wid * rows_w

        def chunk_body(i, carry):
            off = base + i * chunk
            pltpu.sync_copy(uid_hbm.at[pl.ds(off, chunk)], idx1_v)
            pltpu.sync_copy(sid_hbm.at[pl.ds(off, chunk)], idx2_v)
            for kk in range(chunk // 16):
                sl = pl.ds(kk * 16, 16)
                v1 = idx1_v[sl]
                idx1_v[sl] = v1 + v1 - jnp.where(v1 < r1_half, 0, r1 - 1)
                v2 = idx2_v[sl]
                idx2_v[sl] = v2 + v2 - jnp.where(v2 < r2_half, 0, r2 - 1)
            pltpu.async_copy(t1_hbm.at[idx1_v], buf, sem1).wait()
            pltpu.async_copy(t2_hbm.at[idx2_v], buf, sem2, add=True).wait()
            b0 = base_b + i * rows_per_chunk
            for r in range(rows_per_chunk):
                pltpu.sync_copy(buf.at[pl.ds(r * L, L)], out_hbm.at[b0 + r])
            return carry

        lax.fori_loop(0, n_chunks, chunk_body, 0)

    return k(t1, t2, uid, sid)


def kernel(user_id_sequence, skill_sequence, emb1, emb2, W1, b1):
    B, L = user_id_sequence.shape
    E = emb1.shape[1]
    H = W1.shape[1]
    tok = B * L
    r1 = emb1.shape[0]
    r2 = emb2.shape[0]

    b_row = b1.reshape(1, H).astype(jnp.float32)
    zero_row = jnp.zeros((1, H), dtype=jnp.float32)
    t1 = _transform_table(emb1, W1[:E], b_row, blk2=10000).reshape(r1, H)
    t2 = _transform_table(emb2, W1[E:], zero_row, blk2=10000).reshape(r2, H)

    uid = user_id_sequence.reshape(tok).astype(jnp.int32)
    sid = skill_sequence.reshape(tok).astype(jnp.int32)

    return _sc_lookup_sum(
        t1, t2, uid, sid, r1, r2, B, L, H, num_workers=32, rows_per_chunk=4
    )
